# Initial kernel scaffold; baseline (speedup 1.0000x reference)
import jax
import jax.numpy as jnp
from jax.experimental import pallas as pl


def kernel(x, edge_index, W, b):
    raise NotImplementedError("write your pallas kernel here")


# speedup vs baseline: 23.9504x; 23.9504x over previous
import jax
import jax.numpy as jnp
from jax import lax
from jax.experimental import pallas as pl
from jax.experimental.pallas import tpu as pltpu
from jax.experimental.pallas import tpu_sc as plsc

NC = 2
NS = 16
LANES = 16
NW = NC * NS
K = 128
BLK = 256


def _mesh():
    return plsc.VectorSubcoreMesh(core_axis_name="c", subcore_axis_name="s",
                                  num_cores=NC, num_subcores=NS)


def _sc_degree(dst_p, npad, nch):
    rt = npad // NS

    def run(dst_p):
        @pl.kernel(
            out_type=jax.ShapeDtypeStruct((NC, npad, LANES), jnp.float32),
            mesh=_mesh(),
            scratch_types=[
                pltpu.VMEM((nch, K), jnp.int32),
                pltpu.VMEM((K, LANES), jnp.float32),
                pltpu.VMEM_SHARED((npad, LANES), jnp.float32),
            ],
        )
        def deg_kernel(dst_hbm, out_hbm, idx_v, ones_v, acc_sh):
            c = lax.axis_index("c")
            s = lax.axis_index("s")
            wid = c * NS + s
            pltpu.sync_copy(dst_hbm.at[wid], idx_v)

            @pl.loop(0, K)
            def _(r):
                ones_v[pl.ds(r, 1), pl.ds(0, LANES)] = jnp.zeros(
                    (1, LANES), jnp.float32)

            @pl.loop(0, rt, step=K)
            def _(r0):
                pltpu.sync_copy(ones_v, acc_sh.at[pl.ds(s * rt + r0, K)])

            @pl.loop(0, K)
            def _(r):
                ones_v[pl.ds(r, 1), pl.ds(0, LANES)] = jnp.ones(
                    (1, LANES), jnp.float32)

            plsc.subcore_barrier()

            @pl.loop(0, nch)
            def _(j):
                pltpu.sync_copy(ones_v, acc_sh.at[idx_v.at[j]], add=True)

            plsc.subcore_barrier()

            @pl.loop(0, rt, step=K)
            def _(r0):
                pltpu.sync_copy(acc_sh.at[pl.ds(s * rt + r0, K)],
                                out_hbm.at[c, pl.ds(s * rt + r0, K)])

        return deg_kernel(dst_p)

    return run(dst_p)


def _sc_aggregate(g, src_p, dst_p, npad, nch, d):
    rt = npad // NS

    def run(g, src_p, dst_p):
        @pl.kernel(
            out_type=jax.ShapeDtypeStruct((NC, npad, d), jnp.float32),
            mesh=_mesh(),
            scratch_types=[
                pltpu.VMEM((nch, K), jnp.int32),
                pltpu.VMEM((K,), jnp.int32),
                pltpu.VMEM((K, d), jnp.float32),
                pltpu.VMEM_SHARED((npad, d), jnp.float32),
            ],
        )
        def msg_kernel(g_hbm, src_hbm, dst_hbm, out_hbm,
                       dst_v, sia, ra, acc_sh):
            c = lax.axis_index("c")
            s = lax.axis_index("s")
            wid = c * NS + s
            pltpu.sync_copy(dst_hbm.at[wid], dst_v)

            @pl.loop(0, K)
            def _(r):
                @pl.loop(0, d, step=LANES)
                def _(c0):
                    ra[pl.ds(r, 1), pl.ds(c0, LANES)] = jnp.zeros(
                        (1, LANES), jnp.float32)

            @pl.loop(0, rt, step=K)
            def _(r0):
                pltpu.sync_copy(ra, acc_sh.at[pl.ds(s * rt + r0, K)])

            plsc.subcore_barrier()

            @pl.loop(0, nch)
            def _(j):
                pltpu.sync_copy(src_hbm.at[wid, j], sia)
                pltpu.sync_copy(g_hbm.at[sia], ra)
                pltpu.sync_copy(ra, acc_sh.at[dst_v.at[j]], add=True)

            plsc.subcore_barrier()

            @pl.loop(0, rt, step=K)
            def _(r0):
                pltpu.sync_copy(acc_sh.at[pl.ds(s * rt + r0, K)],
                                out_hbm.at[c, pl.ds(s * rt + r0, K)])

        return msg_kernel(g, src_p, dst_p)

    return run(g, src_p, dst_p)


def _deg_dinv(degp_blk):
    deg = jnp.sum(degp_blk, axis=0)
    deg = jnp.sum(deg, axis=1) * (1.0 / LANES) + 1.0
    return lax.rsqrt(deg)


def _tc_linear(x_pad, w, deg_p, npad, d):
    def body(x_ref, w_ref, degp_ref, g_ref):
        dinv = _deg_dinv(degp_ref[...])
        h = jnp.dot(x_ref[...], w_ref[...],
                    preferred_element_type=jnp.float32)
        g_ref[...] = h * dinv[:, None]

    return pl.pallas_call(
        body,
        grid=(npad // BLK,),
        in_specs=[
            pl.BlockSpec((BLK, d), lambda i: (i, 0)),
            pl.BlockSpec((d, d), lambda i: (0, 0)),
            pl.BlockSpec((NC, BLK, LANES), lambda i: (0, i, 0)),
        ],
        out_specs=pl.BlockSpec((BLK, d), lambda i: (i, 0)),
        out_shape=jax.ShapeDtypeStruct((npad, d), jnp.float32),
    )(x_pad, w, deg_p)


def _tc_final(msg_p, g, deg_p, b2, npad, d):
    def body(msgp_ref, g_ref, degp_ref, b_ref, o_ref):
        dinv = _deg_dinv(degp_ref[...])
        tot = jnp.sum(msgp_ref[...], axis=0) + g_ref[...]
        o_ref[...] = jnp.maximum(tot * dinv[:, None] + b_ref[...], 0.0)

    return pl.pallas_call(
        body,
        grid=(npad // BLK,),
        in_specs=[
            pl.BlockSpec((NC, BLK, d), lambda i: (0, i, 0)),
            pl.BlockSpec((BLK, d), lambda i: (i, 0)),
            pl.BlockSpec((NC, BLK, LANES), lambda i: (0, i, 0)),
            pl.BlockSpec((1, d), lambda i: (0, 0)),
        ],
        out_specs=pl.BlockSpec((BLK, d), lambda i: (i, 0)),
        out_shape=jax.ShapeDtypeStruct((npad, d), jnp.float32),
    )(msg_p, g, deg_p, b2)


def kernel(x, edge_index, W, b):
    n, d = x.shape
    e = edge_index.shape[1]

    npad = -(-n // BLK) * BLK
    if npad == n:
        npad += BLK
    ew_real = e // NW
    ew = -(-ew_real // K) * K
    nch = ew // K
    padw = ew - ew_real
    npr = npad - n

    src = edge_index[0].reshape(NW, ew_real).astype(jnp.int32)
    dst = edge_index[1].reshape(NW, ew_real).astype(jnp.int32)
    if padw:
        offs = (jnp.arange(padw)[None, :]
                + (npr // NW) * jnp.arange(NW)[:, None]) % npr
        pad_idx = (n + offs).astype(jnp.int32)
        src = jnp.concatenate([src, pad_idx], axis=1)
        dst = jnp.concatenate([dst, pad_idx], axis=1)
    src_p = src.reshape(NW, nch, K)
    dst_p = dst.reshape(NW, nch, K)

    x_pad = jnp.zeros((npad, d), x.dtype).at[:n].set(x)

    deg_p = _sc_degree(dst_p, npad, nch)
    g = _tc_linear(x_pad, W, deg_p, npad, d)
    msg_p = _sc_aggregate(g, src_p, dst_p, npad, nch, d)
    out = _tc_final(msg_p, g, deg_p, b.reshape(1, d), npad, d)
    return out[:n]


# speedup vs baseline: 30.3043x; 1.2653x over previous
import jax
import jax.numpy as jnp
from jax import lax
from jax.experimental import pallas as pl
from jax.experimental.pallas import tpu as pltpu
from jax.experimental.pallas import tpu_sc as plsc

NC = 2
NS = 16
LANES = 16
NW = NC * NS
K = 128
BLK = 256


def _mesh():
    return plsc.VectorSubcoreMesh(core_axis_name="c", subcore_axis_name="s",
                                  num_cores=NC, num_subcores=NS)


def _sc_degree(dst_p, npad, nch):
    rt = npad // NS

    def run(dst_p):
        @pl.kernel(
            out_type=jax.ShapeDtypeStruct((NC, npad, LANES), jnp.float32),
            mesh=_mesh(),
            scratch_types=[
                pltpu.VMEM((nch, K), jnp.int32),
                pltpu.VMEM((K, LANES), jnp.float32),
                pltpu.VMEM_SHARED((npad, LANES), jnp.float32),
            ],
        )
        def deg_kernel(dst_hbm, out_hbm, idx_v, ones_v, acc_sh):
            c = lax.axis_index("c")
            s = lax.axis_index("s")
            wid = c * NS + s
            pltpu.sync_copy(dst_hbm.at[wid], idx_v)

            @pl.loop(0, K)
            def _(r):
                ones_v[pl.ds(r, 1), pl.ds(0, LANES)] = jnp.zeros(
                    (1, LANES), jnp.float32)

            @pl.loop(0, rt, step=K)
            def _(r0):
                pltpu.sync_copy(ones_v, acc_sh.at[pl.ds(s * rt + r0, K)])

            @pl.loop(0, K)
            def _(r):
                ones_v[pl.ds(r, 1), pl.ds(0, LANES)] = jnp.ones(
                    (1, LANES), jnp.float32)

            plsc.subcore_barrier()

            @pl.loop(0, nch)
            def _(j):
                pltpu.sync_copy(ones_v, acc_sh.at[idx_v.at[j]], add=True)

            plsc.subcore_barrier()

            @pl.loop(0, rt, step=K)
            def _(r0):
                pltpu.sync_copy(acc_sh.at[pl.ds(s * rt + r0, K)],
                                out_hbm.at[c, pl.ds(s * rt + r0, K)])

        return deg_kernel(dst_p)

    return run(dst_p)


def _sc_aggregate(g, src_p, dst_p, npad, nch, d):
    rt = npad // NS

    def run(g, src_p, dst_p):
        @pl.kernel(
            out_type=jax.ShapeDtypeStruct((NC, npad, d), jnp.float32),
            mesh=_mesh(),
            scratch_types=[
                pltpu.VMEM((nch, K), jnp.int32),
                pltpu.VMEM((nch, K), jnp.int32),
                pltpu.VMEM((K, d), jnp.float32),
                pltpu.VMEM_SHARED((npad, d), jnp.float32),
            ],
        )
        def msg_kernel(g_hbm, src_hbm, dst_hbm, out_hbm,
                       dst_v, src_v, ra, acc_sh):
            c = lax.axis_index("c")
            s = lax.axis_index("s")
            wid = c * NS + s
            pltpu.sync_copy(dst_hbm.at[wid], dst_v)
            pltpu.sync_copy(src_hbm.at[wid], src_v)

            @pl.loop(0, K)
            def _(r):
                @pl.loop(0, d, step=LANES)
                def _(c0):
                    ra[pl.ds(r, 1), pl.ds(c0, LANES)] = jnp.zeros(
                        (1, LANES), jnp.float32)

            @pl.loop(0, rt, step=K)
            def _(r0):
                pltpu.sync_copy(ra, acc_sh.at[pl.ds(s * rt + r0, K)])

            plsc.subcore_barrier()

            @pl.loop(0, nch)
            def _(j):
                pltpu.sync_copy(g_hbm.at[src_v.at[j]], ra)
                pltpu.sync_copy(ra, acc_sh.at[dst_v.at[j]], add=True)

            plsc.subcore_barrier()

            @pl.loop(0, rt, step=K)
            def _(r0):
                pltpu.sync_copy(acc_sh.at[pl.ds(s * rt + r0, K)],
                                out_hbm.at[c, pl.ds(s * rt + r0, K)])

        return msg_kernel(g, src_p, dst_p)

    return run(g, src_p, dst_p)


def _deg_dinv(degp_blk):
    deg = jnp.sum(degp_blk, axis=0)
    deg = jnp.sum(deg, axis=1) * (1.0 / LANES) + 1.0
    return lax.rsqrt(deg)


def _tc_linear(x_pad, w, deg_p, npad, d):
    def body(x_ref, w_ref, degp_ref, g_ref):
        dinv = _deg_dinv(degp_ref[...])
        h = jnp.dot(x_ref[...], w_ref[...],
                    preferred_element_type=jnp.float32)
        g_ref[...] = h * dinv[:, None]

    return pl.pallas_call(
        body,
        out_shape=jax.ShapeDtypeStruct((npad, d), jnp.float32),
    )(x_pad, w, deg_p)


def _tc_final(msg_p, g, deg_p, b2, npad, d):
    def body(msgp_ref, g_ref, degp_ref, b_ref, o_ref):
        dinv = _deg_dinv(degp_ref[...])
        tot = jnp.sum(msgp_ref[...], axis=0) + g_ref[...]
        o_ref[...] = jnp.maximum(tot * dinv[:, None] + b_ref[...], 0.0)

    return pl.pallas_call(
        body,
        out_shape=jax.ShapeDtypeStruct((npad, d), jnp.float32),
    )(msg_p, g, deg_p, b2)


def kernel(x, edge_index, W, b):
    n, d = x.shape
    e = edge_index.shape[1]

    npad = -(-n // BLK) * BLK
    if npad == n:
        npad += BLK
    ew_real = e // NW
    ew = -(-ew_real // K) * K
    nch = ew // K
    padw = ew - ew_real
    npr = npad - n

    src = edge_index[0].reshape(NW, ew_real).astype(jnp.int32)
    dst = edge_index[1].reshape(NW, ew_real).astype(jnp.int32)
    if padw:
        offs = (jnp.arange(padw)[None, :]
                + (npr // NW) * jnp.arange(NW)[:, None]) % npr
        pad_idx = (n + offs).astype(jnp.int32)
        src = jnp.concatenate([src, pad_idx], axis=1)
        dst = jnp.concatenate([dst, pad_idx], axis=1)
    src_p = src.reshape(NW, nch, K)
    dst_p = dst.reshape(NW, nch, K)

    x_pad = jnp.zeros((npad, d), x.dtype).at[:n].set(x)

    deg_p = _sc_degree(dst_p, npad, nch)
    g = _tc_linear(x_pad, W, deg_p, npad, d)
    msg_p = _sc_aggregate(g, src_p, dst_p, npad, nch, d)
    out = _tc_final(msg_p, g, deg_p, b.reshape(1, d), npad, d)
    return out[:n]


# speedup vs baseline: 30.9097x; 1.0200x over previous
import jax
import jax.numpy as jnp
from jax import lax
from jax.experimental import pallas as pl
from jax.experimental.pallas import tpu as pltpu
from jax.experimental.pallas import tpu_sc as plsc

NC = 2
NS = 16
LANES = 16
NW = NC * NS
K = 128
BLK = 256


def _mesh():
    return plsc.VectorSubcoreMesh(core_axis_name="c", subcore_axis_name="s",
                                  num_cores=NC, num_subcores=NS)


def _sc_degree(dst_p, npad, nch):
    rt = npad // NS

    def run(dst_p):
        @pl.kernel(
            out_type=jax.ShapeDtypeStruct((NC, npad, LANES), jnp.float32),
            mesh=_mesh(),
            scratch_types=[
                pltpu.VMEM((nch, K), jnp.int32),
                pltpu.VMEM((K, LANES), jnp.float32),
                pltpu.VMEM_SHARED((npad, LANES), jnp.float32),
            ],
        )
        def deg_kernel(dst_hbm, out_hbm, idx_v, ones_v, acc_sh):
            c = lax.axis_index("c")
            s = lax.axis_index("s")
            wid = c * NS + s
            pltpu.sync_copy(dst_hbm.at[wid], idx_v)

            @pl.loop(0, K)
            def _(r):
                ones_v[pl.ds(r, 1), pl.ds(0, LANES)] = jnp.zeros(
                    (1, LANES), jnp.float32)

            @pl.loop(0, rt, step=K)
            def _(r0):
                pltpu.sync_copy(ones_v, acc_sh.at[pl.ds(s * rt + r0, K)])

            @pl.loop(0, K)
            def _(r):
                ones_v[pl.ds(r, 1), pl.ds(0, LANES)] = jnp.ones(
                    (1, LANES), jnp.float32)

            plsc.subcore_barrier()

            @pl.loop(0, nch)
            def _(j):
                pltpu.sync_copy(ones_v, acc_sh.at[idx_v.at[j]], add=True)

            plsc.subcore_barrier()

            @pl.loop(0, rt, step=K)
            def _(r0):
                pltpu.sync_copy(acc_sh.at[pl.ds(s * rt + r0, K)],
                                out_hbm.at[c, pl.ds(s * rt + r0, K)])

        return deg_kernel(dst_p)

    return run(dst_p)


def _sc_aggregate(g, src_p, dst_p, npad, nch, d):
    rt = npad // NS

    def run(g, src_p, dst_p):
        @pl.kernel(
            out_type=jax.ShapeDtypeStruct((NC, npad, d), jnp.float32),
            mesh=_mesh(),
            scratch_types=[
                pltpu.VMEM((nch, K), jnp.int32),
                pltpu.VMEM((nch, K), jnp.int32),
                pltpu.VMEM((K, d), jnp.float32),
                pltpu.VMEM_SHARED((npad, d), jnp.float32),
            ],
        )
        def msg_kernel(g_hbm, src_hbm, dst_hbm, out_hbm,
                       dst_v, src_v, ra, acc_sh):
            c = lax.axis_index("c")
            s = lax.axis_index("s")
            wid = c * NS + s
            pltpu.sync_copy(dst_hbm.at[wid], dst_v)
            pltpu.sync_copy(src_hbm.at[wid], src_v)

            @pl.loop(0, K)
            def _(r):
                @pl.loop(0, d, step=LANES)
                def _(c0):
                    ra[pl.ds(r, 1), pl.ds(c0, LANES)] = jnp.zeros(
                        (1, LANES), jnp.float32)

            @pl.loop(0, rt, step=K)
            def _(r0):
                pltpu.sync_copy(ra, acc_sh.at[pl.ds(s * rt + r0, K)])

            plsc.subcore_barrier()

            @pl.loop(0, nch)
            def _(j):
                pltpu.sync_copy(g_hbm.at[src_v.at[j]], ra)
                pltpu.sync_copy(ra, acc_sh.at[dst_v.at[j]], add=True)

            plsc.subcore_barrier()

            @pl.loop(0, rt, step=K)
            def _(r0):
                pltpu.sync_copy(acc_sh.at[pl.ds(s * rt + r0, K)],
                                out_hbm.at[c, pl.ds(s * rt + r0, K)])

        return msg_kernel(g, src_p, dst_p)

    return run(g, src_p, dst_p)


def _deg_dinv(degp_blk):
    deg = jnp.sum(degp_blk, axis=0)
    deg = jnp.sum(deg, axis=1) * (1.0 / LANES) + 1.0
    return lax.rsqrt(deg)


def _tc_linear(x, w, deg_p, npad, d):
    n = x.shape[0]

    def body(x_ref, w_ref, degp_ref, g_ref):
        dinv = _deg_dinv(degp_ref[...])[:n]
        h = jnp.dot(x_ref[...], w_ref[...],
                    preferred_element_type=jnp.float32)
        g_ref[pl.ds(0, n)] = h * dinv[:, None]
        g_ref[pl.ds(n, npad - n)] = jnp.zeros((npad - n, d), jnp.float32)

    return pl.pallas_call(
        body,
        out_shape=jax.ShapeDtypeStruct((npad, d), jnp.float32),
    )(x, w, deg_p)


def _tc_final(msg_p, g, deg_p, b2, n, d):
    def body(msgp_ref, g_ref, degp_ref, b_ref, o_ref):
        dinv = _deg_dinv(degp_ref[...])[:n]
        tot = (msgp_ref[0, pl.ds(0, n)] + msgp_ref[1, pl.ds(0, n)]
               + g_ref[pl.ds(0, n)])
        o_ref[...] = jnp.maximum(tot * dinv[:, None] + b_ref[...], 0.0)

    return pl.pallas_call(
        body,
        out_shape=jax.ShapeDtypeStruct((n, d), jnp.float32),
    )(msg_p, g, deg_p, b2)


def kernel(x, edge_index, W, b):
    n, d = x.shape
    e = edge_index.shape[1]

    npad = -(-n // BLK) * BLK
    if npad == n:
        npad += BLK
    ew_real = e // NW
    ew = -(-ew_real // K) * K
    nch = ew // K
    padw = ew - ew_real
    npr = npad - n

    src = edge_index[0].reshape(NW, ew_real).astype(jnp.int32)
    dst = edge_index[1].reshape(NW, ew_real).astype(jnp.int32)
    if padw:
        offs = (jnp.arange(padw)[None, :]
                + (npr // NW) * jnp.arange(NW)[:, None]) % npr
        pad_idx = (n + offs).astype(jnp.int32)
        src = jnp.concatenate([src, pad_idx], axis=1)
        dst = jnp.concatenate([dst, pad_idx], axis=1)
    src_p = src.reshape(NW, nch, K)
    dst_p = dst.reshape(NW, nch, K)

    deg_p = _sc_degree(dst_p, npad, nch)
    g = _tc_linear(x, W, deg_p, npad, d)
    msg_p = _sc_aggregate(g, src_p, dst_p, npad, nch, d)
    return _tc_final(msg_p, g, deg_p, b.reshape(1, d), n, d)
